# baseline (speedup 1.0000x reference)
import dataclasses
import functools

import jax
import jax.numpy as jnp
from jax import lax
from jax.experimental import pallas as pl
from jax.experimental.pallas import tpu as pltpu
from jax.experimental.pallas import tpu_sc as plsc

N = 10000
E = 320000
F = 128
H = 128
C = 64

NC = 2
NS = 16
NW = NC * NS

N_PAD = 10240
RPT = N_PAD // NS

S_SPLIT = 5000
ACC_ROWS = 5120
RPT2 = ACC_ROWS // NS
PAD_ROW = 5056

CHUNK = 80
EPT = E // NW
NCH_E = EPT // CHUNK
CAP = 5440
NCH_B = CAP // CHUNK
NCH_HOP = 2 * NCH_B
NCHUNK_DEG = 2 * NCH_E

_MESH = plsc.VectorSubcoreMesh(
    core_axis_name="c", subcore_axis_name="s", num_cores=NC, num_subcores=NS
)

_SC_PARAMS = pltpu.CompilerParams()
if "needs_layout_passes" in pltpu.CompilerParams.__dataclass_fields__:
    _SC_PARAMS = dataclasses.replace(_SC_PARAMS, needs_layout_passes=False)
if "use_tc_tiling_on_sc" in pltpu.CompilerParams.__dataclass_fields__:
    _SC_PARAMS = dataclasses.replace(_SC_PARAMS, use_tc_tiling_on_sc=False)


def _rsqrt_newton(x):
    i = plsc.bitcast(x, jnp.int32)
    i = jnp.int32(0x5F3759DF) - lax.shift_right_logical(i, 1)
    y = plsc.bitcast(i, jnp.float32)
    for _ in range(3):
        y = y * (1.5 - 0.5 * x * y * y)
    return y


_i32b = jax.ShapeDtypeStruct((NW, 2, NCH_B, CHUNK), jnp.int32)


@functools.partial(
    pl.kernel,
    out_type=(jax.ShapeDtypeStruct((N_PAD,), jnp.float32), _i32b, _i32b),
    mesh=_MESH,
    scratch_types=[
        pltpu.VMEM((NCHUNK_DEG, CHUNK), jnp.int32),
        pltpu.VMEM((CHUNK,), jnp.float32),
        pltpu.VMEM((RPT,), jnp.float32),
        pltpu.VMEM((NCH_E, CHUNK), jnp.int32),
        pltpu.VMEM((NCH_E, CHUNK), jnp.int32),
        pltpu.VMEM((NCH_B, CHUNK), jnp.int32),
        pltpu.VMEM((NCH_B, CHUNK), jnp.int32),
        pltpu.VMEM((NCH_B, CHUNK), jnp.int32),
        pltpu.VMEM((NCH_B, CHUNK), jnp.int32),
        pltpu.SMEM((2,), jnp.int32),
        pltpu.VMEM_SHARED((N_PAD,), jnp.float32),
        pltpu.SemaphoreType.DMA,
    ],
    compiler_params=_SC_PARAMS,
)
def _deg_bucket_kernel(esrc_hbm, edst_hbm, zeros_hbm, ones_hbm,
                       dummy0_hbm, dummyp_hbm,
                       norm_hbm, bsrc_hbm, bdst_hbm,
                       dstdeg_v, ones_v, slice_v, esrc_v, edst_v,
                       bsl_v, bdl_v, bsh_v, bdh_v, off_sm, acc_sh, sem):
    c = lax.axis_index("c")
    s = lax.axis_index("s")
    w = c * NS + s

    pltpu.sync_copy(esrc_hbm.at[w], esrc_v)
    pltpu.sync_copy(edst_hbm.at[w], edst_v)
    pltpu.sync_copy(dummy0_hbm, bsl_v)
    pltpu.sync_copy(dummy0_hbm, bsh_v)
    pltpu.sync_copy(dummyp_hbm, bdl_v)
    pltpu.sync_copy(dummyp_hbm, bdh_v)
    off_sm[0] = 0
    off_sm[1] = 0

    @pl.loop(0, EPT, step=16)
    def _(i):
        r = i // CHUNK
        o = i % CHUNK
        es = esrc_v[r, pl.ds(o, 16)]
        ed = edst_v[r, pl.ds(o, 16)]
        is_lo = ed < S_SPLIT
        m1 = jnp.where(is_lo, 1, 0)
        m2 = 1 - m1
        pos = off_sm[0] + plsc.cumsum(m1) - m1
        mk = jnp.logical_and(is_lo, pos < CAP)
        plsc.store_scatter(bsl_v, [pos // CHUNK, pos % CHUNK], es, mask=mk)
        plsc.store_scatter(bdl_v, [pos // CHUNK, pos % CHUNK], ed, mask=mk)
        off_sm[0] = off_sm[0] + jnp.sum(m1)
        posh = off_sm[1] + plsc.cumsum(m2) - m2
        mh = jnp.logical_and(jnp.logical_not(is_lo), posh < CAP)
        plsc.store_scatter(bsh_v, [posh // CHUNK, posh % CHUNK], es, mask=mh)
        plsc.store_scatter(bdh_v, [posh // CHUNK, posh % CHUNK],
                           ed - S_SPLIT, mask=mh)
        off_sm[1] = off_sm[1] + jnp.sum(m2)

    pltpu.sync_copy(bsl_v, bsrc_hbm.at[w, 0])
    pltpu.sync_copy(bdl_v, bdst_hbm.at[w, 0])
    pltpu.sync_copy(bsh_v, bsrc_hbm.at[w, 1])
    pltpu.sync_copy(bdh_v, bdst_hbm.at[w, 1])

    @pl.when(c == 0)
    def _():
        base = s * RPT
        pltpu.sync_copy(zeros_hbm, acc_sh.at[pl.ds(base, RPT)])
        pltpu.sync_copy(edst_hbm.at[2 * s], dstdeg_v.at[pl.ds(0, NCH_E)])
        pltpu.sync_copy(edst_hbm.at[2 * s + 1],
                        dstdeg_v.at[pl.ds(NCH_E, NCH_E)])
        pltpu.sync_copy(ones_hbm, ones_v)
        plsc.subcore_barrier()

        @pl.loop(0, NCHUNK_DEG, step=10)
        def _(j):
            for t in range(10):
                pltpu.async_copy(ones_v, acc_sh.at[dstdeg_v.at[j + t]], sem,
                                 add=True)
            for t in range(10):
                pltpu.make_async_copy(ones_v, acc_sh.at[dstdeg_v.at[j + t]],
                                      sem).wait()

        plsc.subcore_barrier()
        pltpu.sync_copy(acc_sh.at[pl.ds(base, RPT)], slice_v)

        @pl.loop(0, RPT, step=16)
        def _(k):
            d = slice_v[pl.ds(k, 16)]
            slice_v[pl.ds(k, 16)] = jnp.where(d > 0.5, _rsqrt_newton(d), 0.0)

        pltpu.sync_copy(slice_v, norm_hbm.at[pl.ds(base, RPT)])


RING = 3
PRE = 2
_MAIN_END = ((NCH_HOP - PRE) // RING) * RING


@functools.partial(
    pl.kernel,
    out_type=jax.ShapeDtypeStruct((NC, ACC_ROWS, F), jnp.float32),
    mesh=_MESH,
    scratch_types=[
        pltpu.VMEM((NCH_HOP, CHUNK), jnp.int32),
        pltpu.VMEM((NCH_HOP, CHUNK), jnp.int32),
        [pltpu.VMEM((CHUNK, F), jnp.float32)] * RING,
        pltpu.VMEM_SHARED((ACC_ROWS, F), jnp.float32),
        [pltpu.SemaphoreType.DMA] * RING,
        [pltpu.SemaphoreType.DMA] * RING,
    ],
    compiler_params=_SC_PARAMS,
)
def _hop_kernel(g_hbm, bsrc_hbm, bdst_hbm, zeros_hbm, out_hbm,
                src_v, dst_v, bufs, acc_sh, gsems, ssems):
    c = lax.axis_index("c")
    s = lax.axis_index("s")
    base = s * RPT2

    def fire_gather(k, b):
        pltpu.async_copy(g_hbm.at[src_v.at[k]], bufs[b], gsems[b])

    def wait_gather(k, b):
        pltpu.make_async_copy(g_hbm.at[src_v.at[k]], bufs[b], gsems[b]).wait()

    def fire_scatter(k, b):
        pltpu.async_copy(bufs[b], acc_sh.at[dst_v.at[k]], ssems[b], add=True)

    def wait_scatter(k, b):
        pltpu.make_async_copy(bufs[b], acc_sh.at[dst_v.at[k]],
                              ssems[b]).wait()

    pltpu.sync_copy(bsrc_hbm.at[2 * s, c], src_v.at[pl.ds(0, NCH_B)])
    pltpu.sync_copy(bsrc_hbm.at[2 * s + 1, c], src_v.at[pl.ds(NCH_B, NCH_B)])
    pltpu.sync_copy(bdst_hbm.at[2 * s, c], dst_v.at[pl.ds(0, NCH_B)])
    pltpu.sync_copy(bdst_hbm.at[2 * s + 1, c], dst_v.at[pl.ds(NCH_B, NCH_B)])
    for k in range(PRE):
        fire_gather(k, k % RING)
    pltpu.sync_copy(zeros_hbm, acc_sh.at[pl.ds(base, RPT2)])
    plsc.subcore_barrier()

    def step(k):
        wait_gather(k, k % RING)
        fire_scatter(k, k % RING)
        p = k + PRE
        if p < NCH_HOP:
            q = p - RING
            if q >= 0:
                wait_scatter(q, p % RING)
            fire_gather(p, p % RING)

    for k in range(RING):
        step(k)

    @pl.loop(RING, _MAIN_END, step=RING)
    def _(j):
        for b in range(RING):
            k = j + b
            wait_gather(k, b)
            fire_scatter(k, b)
            wait_scatter(k + PRE - RING, (b + PRE) % RING)
            fire_gather(k + PRE, (b + PRE) % RING)

    for k in range(_MAIN_END, NCH_HOP):
        step(k)
    for k in range(NCH_HOP - RING, NCH_HOP):
        wait_scatter(k, k % RING)

    plsc.subcore_barrier()
    pltpu.sync_copy(acc_sh.at[pl.ds(base, RPT2)],
                    out_hbm.at[c, pl.ds(base, RPT2)])


BR = 1000
NBLK = N // BR
HBLK = S_SPLIT // BR


def _dot_t(a, b):
    return lax.dot_general(a, b, (((1,), (1,)), ((), ())),
                           precision=lax.Precision.HIGHEST,
                           preferred_element_type=jnp.float32)


def _fold_body(wsgc_ref, wlin_ref, wproj_ref, w1_ref, w2_ref):
    dn = (((1,), (0,)), ((), ()))
    w1_ref[...] = lax.dot_general(
        wproj_ref[:, :H], wlin_ref[...], dn,
        precision=lax.Precision.HIGHEST, preferred_element_type=jnp.float32)
    w2_ref[...] = lax.dot_general(
        wproj_ref[:, H:], wsgc_ref[...], dn,
        precision=lax.Precision.HIGHEST, preferred_element_type=jnp.float32)


def _root_body(feat_ref, w1_ref, out_ref):
    out_ref[...] = _dot_t(feat_ref[...], w1_ref[...])


def _prep_body(feat_ref, norm_ref, out_ref):
    out_ref[...] = feat_ref[...] * norm_ref[...]


def _mid_body(p_ref, norm_ref, out_ref):
    n2 = norm_ref[...] * norm_ref[...]
    out_ref[...] = p_ref[0] * n2


def _final_body(p_ref, norm_ref, x1p_ref, w2_ref, out_ref):
    s2 = p_ref[0] * norm_ref[...]
    out_ref[...] = x1p_ref[...] + _dot_t(s2, w2_ref[...])


_full_spec = pl.BlockSpec((BR, F), lambda i: (i, 0))
_norm_spec = pl.BlockSpec((BR, 1), lambda i: (i, 0))
_p_spec = pl.BlockSpec((1, BR, F), lambda b: (b // HBLK, b % HBLK, 0))
_out_spec = pl.BlockSpec((BR, C), lambda i: (i, 0))


def _w_spec(r, c_):
    return pl.BlockSpec((r, c_), lambda i: (0, 0))


_wcf = jax.ShapeDtypeStruct((C, F), jnp.float32)
_fold = pl.pallas_call(_fold_body, out_shape=[_wcf, _wcf])
_root = pl.pallas_call(
    _root_body, out_shape=jax.ShapeDtypeStruct((N, C), jnp.float32),
    grid=(NBLK,), in_specs=[_full_spec, _w_spec(C, F)], out_specs=_out_spec)
_prep = pl.pallas_call(
    _prep_body, out_shape=jax.ShapeDtypeStruct((N, F), jnp.float32),
    grid=(NBLK,), in_specs=[_full_spec, _norm_spec], out_specs=_full_spec)
_mid = pl.pallas_call(
    _mid_body, out_shape=jax.ShapeDtypeStruct((N, F), jnp.float32),
    grid=(NBLK,), in_specs=[_p_spec, _norm_spec], out_specs=_full_spec)
_final = pl.pallas_call(
    _final_body, out_shape=jax.ShapeDtypeStruct((N, C), jnp.float32),
    grid=(NBLK,),
    in_specs=[_p_spec, _norm_spec, _out_spec, _w_spec(C, F)],
    out_specs=_out_spec)


@jax.jit
def kernel(features, edge_index, W_sgc, W_lin, W_proj):
    src = edge_index[0].astype(jnp.int32)
    dst = edge_index[1].astype(jnp.int32)
    esrc = src.reshape(NW, NCH_E, CHUNK)
    edst = dst.reshape(NW, NCH_E, CHUNK)

    zeros_deg = jnp.zeros((RPT,), jnp.float32)
    ones_chunk = jnp.ones((CHUNK,), jnp.float32)
    dummy0 = jnp.zeros((NCH_B, CHUNK), jnp.int32)
    dummyp = jnp.full((NCH_B, CHUNK), PAD_ROW, jnp.int32)
    zeros_rows = jnp.zeros((RPT2, F), jnp.float32)

    w1, w2 = _fold(W_sgc, W_lin, W_proj)
    x1p = _root(features, w1)

    norm_flat, bsrc, bdst = _deg_bucket_kernel(
        esrc, edst, zeros_deg, ones_chunk, dummy0, dummyp)
    norm1 = norm_flat[:N].reshape(N, 1)

    g1 = _prep(features, norm1)
    p1 = _hop_kernel(g1, bsrc, bdst, zeros_rows)
    g2 = _mid(p1, norm1)
    p2 = _hop_kernel(g2, bsrc, bdst, zeros_rows)

    return _final(p2, norm1, x1p, w2)
